# baseline (speedup 1.0000x reference)
import functools

import jax
import jax.numpy as jnp
from jax import lax
from jax.experimental import pallas as pl
from jax.experimental.pallas import tpu as pltpu
from jax.experimental.pallas import tpu_sc as plsc

N_SRC = 5000
N_OUT = 10000
D = 512
NW = 32
REG = 312
REG_LAST = N_OUT - (NW - 1) * REG
CH = 24
NCH = REG // CH
TAIL = REG_LAST - REG
IDX_PAD = 8208
SENTINEL = 1 << 20
NPAD = 512
SRC_LEN = 336


def _unpool_body(x_hbm, idx_hbm, out_hbm, idx_v, src_v, rows_a, rows_b,
                 rows_t, sem_a, sem_b, sem_t):
    cid = lax.axis_index("c")
    sid = lax.axis_index("s")
    wid = sid * 2 + cid
    r0 = pl.multiple_of(wid * REG, 8)
    is_last = wid == NW - 1
    reg = jnp.where(is_last, REG_LAST, REG)

    with jax.named_scope("stage_idx"):
        pltpu.sync_copy(idx_hbm, idx_v)

    lanes0 = lax.iota(jnp.int32, 16)
    for j in range(SRC_LEN // 16):
        zfill = N_SRC + ((wid * 16 + j * 16 + lanes0) & (NPAD - 1))
        src_v[pl.ds(j * 16, 16)] = zfill

    def lower_bound(target):
        lo = jnp.int32(0)
        for s in (4096, 2048, 1024, 512, 256, 128, 64, 32, 16, 8, 4, 2, 1):
            v = idx_v[pl.ds(lo + s - 1, 16)][0]
            lo = jnp.where(v < target, lo + s, lo)
        return lo

    e_lo = lower_bound(r0)
    e_hi = lower_bound(r0 + reg)

    lanes = lax.iota(jnp.int32, 16)

    def scan_step(k, carry):
        off = k * 16
        a = idx_v[pl.ds(off, 16)]
        b = idx_v[pl.ds(off + 1, 16)]
        m = (a != b) & (a >= r0) & (a < r0 + reg)
        plsc.store_scatter(src_v, [a - r0], lanes + off, mask=m)
        return carry

    with jax.named_scope("scan"):
        lax.fori_loop(e_lo // 16, (e_hi + 15) // 16, scan_step, 0)

    def gather(c, buf, sem):
        return pltpu.make_async_copy(
            x_hbm.at[src_v.at[pl.ds(c * CH, CH)]], buf, sem)

    with jax.named_scope("pull"):
        bufs = (rows_a, rows_b)
        sems = (sem_a, sem_b)
        gather(0, rows_a, sem_a).start()

        @pl.when(is_last)
        def _tail_start():
            pltpu.make_async_copy(
                x_hbm.at[src_v.at[pl.ds(NCH * CH, TAIL)]], rows_t, sem_t,
            ).start()

        for c in range(NCH):
            cur = c % 2
            if c + 1 < NCH:
                gather(c + 1, bufs[1 - cur], sems[1 - cur]).start()
            gather(c, bufs[cur], sems[cur]).wait()
            pltpu.sync_copy(bufs[cur], out_hbm.at[pl.ds(r0 + c * CH, CH)])

        @pl.when(is_last)
        def _tail_drain():
            pltpu.make_async_copy(
                x_hbm.at[src_v.at[pl.ds(NCH * CH, TAIL)]], rows_t, sem_t,
            ).wait()
            pltpu.sync_copy(rows_t, out_hbm.at[pl.ds(r0 + NCH * CH, TAIL)])


_unpool = functools.partial(
    pl.kernel,
    out_type=jax.ShapeDtypeStruct((N_OUT, D), jnp.float32),
    mesh=plsc.VectorSubcoreMesh(core_axis_name="c", subcore_axis_name="s"),
    compiler_params=pltpu.CompilerParams(needs_layout_passes=False),
    scratch_types=[
        pltpu.VMEM((IDX_PAD,), jnp.int32),
        pltpu.VMEM((SRC_LEN,), jnp.int32),
        pltpu.VMEM((CH, D), jnp.float32),
        pltpu.VMEM((CH, D), jnp.float32),
        pltpu.VMEM((TAIL, D), jnp.float32),
        pltpu.SemaphoreType.DMA,
        pltpu.SemaphoreType.DMA,
        pltpu.SemaphoreType.DMA,
    ],
)(_unpool_body)


UPN = 10000
UPB = 80


def _copy_body(a_ref, o_ref):
    o_ref[...] = a_ref[...]


_up_copy = pl.pallas_call(
    _copy_body,
    out_shape=jax.ShapeDtypeStruct((UPN, UPN), jnp.float32),
    grid=(UPN // UPB,),
    in_specs=[pl.BlockSpec((UPB, UPN), lambda i: (i, 0))],
    out_specs=pl.BlockSpec((UPB, UPN), lambda i: (i, 0)),
)


def kernel(x, A, up_A, idx):
    x_pad = jnp.concatenate([x, jnp.zeros((NPAD, D), x.dtype)], axis=0)
    idx_pad = jnp.concatenate([
        idx.astype(jnp.int32),
        jnp.full((IDX_PAD - N_SRC,), SENTINEL, jnp.int32),
    ])
    new_x = _unpool(x_pad, idx_pad)
    return (new_x, _up_copy(up_A))


# speedup vs baseline: 1.0216x; 1.0216x over previous
import functools

import jax
import jax.numpy as jnp
from jax import lax
from jax.experimental import pallas as pl
from jax.experimental.pallas import tpu as pltpu
from jax.experimental.pallas import tpu_sc as plsc

N_SRC = 5000
N_OUT = 10000
D = 512
NW = 32
REG = 312
REG_LAST = N_OUT - (NW - 1) * REG
CH = 24
NCH = REG // CH
TAIL = REG_LAST - REG
IDX_PAD = 8208
SENTINEL = 1 << 20
NPAD = 512
SRC_LEN = 336


def _unpool_body(x_hbm, idx_hbm, out_hbm, idx_v, src_v, rows_a, rows_b,
                 rows_t, sem_a, sem_b, sem_t):
    cid = lax.axis_index("c")
    sid = lax.axis_index("s")
    wid = sid * 2 + cid
    r0 = pl.multiple_of(wid * REG, 8)
    is_last = wid == NW - 1
    reg = jnp.where(is_last, REG_LAST, REG)

    with jax.named_scope("stage_idx"):
        pltpu.sync_copy(idx_hbm, idx_v)

    lanes0 = lax.iota(jnp.int32, 16)
    for j in range(SRC_LEN // 16):
        zfill = N_SRC + ((wid * 16 + j * 16 + lanes0) & (NPAD - 1))
        src_v[pl.ds(j * 16, 16)] = zfill

    def lower_bound(target):
        lo = jnp.int32(0)
        for s in (4096, 2048, 1024, 512, 256, 128, 64, 32, 16, 8, 4, 2, 1):
            v = idx_v[pl.ds(lo + s - 1, 16)][0]
            lo = jnp.where(v < target, lo + s, lo)
        return lo

    e_lo = lower_bound(r0)
    e_hi = lower_bound(r0 + reg)

    lanes = lax.iota(jnp.int32, 16)

    def scan_step(k, carry):
        off = k * 16
        a = idx_v[pl.ds(off, 16)]
        b = idx_v[pl.ds(off + 1, 16)]
        m = (a != b) & (a >= r0) & (a < r0 + reg)
        plsc.store_scatter(src_v, [a - r0], lanes + off, mask=m)
        return carry

    with jax.named_scope("scan"):
        lax.fori_loop(e_lo // 16, (e_hi + 15) // 16, scan_step, 0)

    def gather(c, buf, sem):
        return pltpu.make_async_copy(
            x_hbm.at[src_v.at[pl.ds(c * CH, CH)]], buf, sem)

    with jax.named_scope("pull"):
        bufs = (rows_a, rows_b)
        sems = (sem_a, sem_b)
        gather(0, rows_a, sem_a).start()

        @pl.when(is_last)
        def _tail_start():
            pltpu.make_async_copy(
                x_hbm.at[src_v.at[pl.ds(NCH * CH, TAIL)]], rows_t, sem_t,
            ).start()

        for c in range(NCH):
            cur = c % 2
            if c + 1 < NCH:
                gather(c + 1, bufs[1 - cur], sems[1 - cur]).start()
            gather(c, bufs[cur], sems[cur]).wait()
            pltpu.sync_copy(bufs[cur], out_hbm.at[pl.ds(r0 + c * CH, CH)])

        @pl.when(is_last)
        def _tail_drain():
            pltpu.make_async_copy(
                x_hbm.at[src_v.at[pl.ds(NCH * CH, TAIL)]], rows_t, sem_t,
            ).wait()
            pltpu.sync_copy(rows_t, out_hbm.at[pl.ds(r0 + NCH * CH, TAIL)])


_unpool = functools.partial(
    pl.kernel,
    out_type=jax.ShapeDtypeStruct((N_OUT, D), jnp.float32),
    mesh=plsc.VectorSubcoreMesh(core_axis_name="c", subcore_axis_name="s"),
    compiler_params=pltpu.CompilerParams(needs_layout_passes=False),
    scratch_types=[
        pltpu.VMEM((IDX_PAD,), jnp.int32),
        pltpu.VMEM((SRC_LEN,), jnp.int32),
        pltpu.VMEM((CH, D), jnp.float32),
        pltpu.VMEM((CH, D), jnp.float32),
        pltpu.VMEM((TAIL, D), jnp.float32),
        pltpu.SemaphoreType.DMA,
        pltpu.SemaphoreType.DMA,
        pltpu.SemaphoreType.DMA,
    ],
)(_unpool_body)


UPN = 10000
UPB = 200


def _copy_body(a_ref, o_ref):
    pltpu.sync_copy(a_ref, o_ref)


_up_copy = pl.pallas_call(
    _copy_body,
    out_shape=jax.ShapeDtypeStruct((UPN, UPN), jnp.float32),
    grid=(UPN // UPB,),
    in_specs=[pl.BlockSpec((UPB, UPN), lambda i: (i, 0))],
    out_specs=pl.BlockSpec((UPB, UPN), lambda i: (i, 0)),
)


def kernel(x, A, up_A, idx):
    x_pad = jnp.concatenate([x, jnp.zeros((NPAD, D), x.dtype)], axis=0)
    idx_pad = jnp.concatenate([
        idx.astype(jnp.int32),
        jnp.full((IDX_PAD - N_SRC,), SENTINEL, jnp.int32),
    ])
    new_x = _unpool(x_pad, idx_pad)
    return (new_x, _up_copy(up_A))


# speedup vs baseline: 1.0227x; 1.0011x over previous
import functools

import jax
import jax.numpy as jnp
from jax import lax
from jax.experimental import pallas as pl
from jax.experimental.pallas import tpu as pltpu
from jax.experimental.pallas import tpu_sc as plsc

N_SRC = 5000
N_OUT = 10000
D = 512
NW = 32
REG = 312
REG_LAST = N_OUT - (NW - 1) * REG
CH = 24
NCH = REG // CH
TAIL = REG_LAST - REG
IDX_PAD = 8208
SENTINEL = 1 << 20
NPAD = 512
SRC_LEN = 336


def _unpool_body(x_hbm, idx_hbm, out_hbm, idx_v, src_v, rows_a, rows_b,
                 rows_t, sem_a, sem_b, sem_t):
    cid = lax.axis_index("c")
    sid = lax.axis_index("s")
    wid = sid * 2 + cid
    r0 = pl.multiple_of(wid * REG, 8)
    is_last = wid == NW - 1
    reg = jnp.where(is_last, REG_LAST, REG)

    with jax.named_scope("stage_idx"):
        pltpu.sync_copy(idx_hbm, idx_v)

    lanes0 = lax.iota(jnp.int32, 16)
    for j in range(SRC_LEN // 16):
        zfill = N_SRC + ((wid * 16 + j * 16 + lanes0) & (NPAD - 1))
        src_v[pl.ds(j * 16, 16)] = zfill

    def lower_bound(target):
        lo = jnp.int32(0)
        for s in (4096, 2048, 1024, 512, 256, 128, 64, 32, 16, 8, 4, 2, 1):
            v = idx_v[pl.ds(lo + s - 1, 16)][0]
            lo = jnp.where(v < target, lo + s, lo)
        return lo

    e_lo = lower_bound(r0)
    e_hi = lower_bound(r0 + reg)

    lanes = lax.iota(jnp.int32, 16)

    def scan_step(k, carry):
        off = k * 16
        a = idx_v[pl.ds(off, 16)]
        b = idx_v[pl.ds(off + 1, 16)]
        m = (a != b) & (a >= r0) & (a < r0 + reg)
        plsc.store_scatter(src_v, [a - r0], lanes + off, mask=m)
        return carry

    with jax.named_scope("scan"):
        lax.fori_loop(e_lo // 16, (e_hi + 15) // 16, scan_step, 0)

    def gather(c, buf, sem):
        return pltpu.make_async_copy(
            x_hbm.at[src_v.at[pl.ds(c * CH, CH)]], buf, sem)

    with jax.named_scope("pull"):
        bufs = (rows_a, rows_b)
        sems = (sem_a, sem_b)
        gather(0, rows_a, sem_a).start()

        @pl.when(is_last)
        def _tail_start():
            pltpu.make_async_copy(
                x_hbm.at[src_v.at[pl.ds(NCH * CH, TAIL)]], rows_t, sem_t,
            ).start()

        for c in range(NCH):
            cur = c % 2
            if c + 1 < NCH:
                gather(c + 1, bufs[1 - cur], sems[1 - cur]).start()
            gather(c, bufs[cur], sems[cur]).wait()
            pltpu.sync_copy(bufs[cur], out_hbm.at[pl.ds(r0 + c * CH, CH)])

        @pl.when(is_last)
        def _tail_drain():
            pltpu.make_async_copy(
                x_hbm.at[src_v.at[pl.ds(NCH * CH, TAIL)]], rows_t, sem_t,
            ).wait()
            pltpu.sync_copy(rows_t, out_hbm.at[pl.ds(r0 + NCH * CH, TAIL)])


_unpool = functools.partial(
    pl.kernel,
    out_type=jax.ShapeDtypeStruct((N_OUT, D), jnp.float32),
    mesh=plsc.VectorSubcoreMesh(core_axis_name="c", subcore_axis_name="s"),
    compiler_params=pltpu.CompilerParams(needs_layout_passes=False),
    scratch_types=[
        pltpu.VMEM((IDX_PAD,), jnp.int32),
        pltpu.VMEM((SRC_LEN,), jnp.int32),
        pltpu.VMEM((CH, D), jnp.float32),
        pltpu.VMEM((CH, D), jnp.float32),
        pltpu.VMEM((TAIL, D), jnp.float32),
        pltpu.SemaphoreType.DMA,
        pltpu.SemaphoreType.DMA,
        pltpu.SemaphoreType.DMA,
    ],
)(_unpool_body)


UPN = 10000
UPB = 200


def _copy_body(a_ref, o_ref):
    o_ref[...] = a_ref[...]


_up_copy = pl.pallas_call(
    _copy_body,
    out_shape=jax.ShapeDtypeStruct((UPN, UPN), jnp.float32),
    grid=(UPN // UPB,),
    in_specs=[pl.BlockSpec((UPB, UPN), lambda i: (i, 0))],
    out_specs=pl.BlockSpec((UPB, UPN), lambda i: (i, 0)),
)


def kernel(x, A, up_A, idx):
    x_pad = jnp.concatenate([x, jnp.zeros((NPAD, D), x.dtype)], axis=0)
    idx_pad = jnp.concatenate([
        idx.astype(jnp.int32),
        jnp.full((IDX_PAD - N_SRC,), SENTINEL, jnp.int32),
    ])
    new_x = _unpool(x_pad, idx_pad)
    return (new_x, _up_copy(up_A))
